# baseline (speedup 1.0000x reference)
import functools
import jax
import jax.numpy as jnp
from jax import lax
from jax.experimental import pallas as pl
from jax.experimental.pallas import tpu as pltpu
from jax.experimental.pallas import tpu_sc as plsc

R = 1024
N = 32768
L = 16
NB = 2048
VREGS = N // L
HVREGS = VREGS // 2
HALF = N // 2
MIN32 = jnp.int32(-2147483648)
MASK11 = jnp.int32(0x7FF)


@functools.cache
def _build_sort_kernel():
    info = plsc.get_sparse_core_info()
    nw = info.num_cores * info.num_subcores
    assert R % nw == 0
    rows_per_w = R // nw
    mesh = plsc.VectorSubcoreMesh(core_axis_name="c", subcore_axis_name="s")

    @functools.partial(
        pl.kernel,
        out_type=[
            jax.ShapeDtypeStruct((R, N), jnp.int32),
            jax.ShapeDtypeStruct((R, N), jnp.int32),
        ],
        mesh=mesh,
        compiler_params=pltpu.CompilerParams(needs_layout_passes=False),
        scratch_types=[
            pltpu.VMEM((N,), jnp.int32),
            pltpu.VMEM((N,), jnp.int32),
            pltpu.VMEM((N,), jnp.int32),
            pltpu.VMEM((NB,), jnp.int32),
            pltpu.VMEM((NB,), jnp.int32),
            pltpu.VMEM((NB,), jnp.int32),
            pltpu.VMEM((NB,), jnp.int32),
        ],
    )
    def sort_kernel(xbits_hbm, vals_hbm, idx_hbm, kbuf, abuf, bbuf, h0, h1, o0, o1):
        wid = lax.axis_index("s") * info.num_cores + lax.axis_index("c")
        iota = lax.iota(jnp.int32, L)
        zeros = jnp.zeros((L,), jnp.int32)

        def clear_body(j, _):
            h0[pl.ds(j * L, L)] = zeros
            h1[pl.ds(j * L, L)] = zeros
            return 0

        lax.fori_loop(0, NB // L, clear_body, 0)

        def hist_update(h, dig, mask=None):
            cnt, lastm = plsc.scan_count(dig, mask=mask)
            plsc.addupdate_scatter(h, [dig], cnt, mask=lastm)

        def row_body(r, _):
            row = wid * rows_per_w + r
            pltpu.sync_copy(xbits_hbm.at[row], bbuf)

            def conv_body(v, _):
                for h, vv in ((h0, v), (h1, v + HVREGS)):
                    b = bbuf[pl.ds(vv * L, L)]
                    m = jnp.where(b >= 0, b ^ MIN32, ~b)
                    kbuf[pl.ds(vv * L, L)] = m
                    hist_update(h, m & MASK11)
                return 0

            lax.fori_loop(0, HVREGS, conv_body, 0)

            def scan_body(j, carry):
                ha = h0[pl.ds(j * L, L)]
                hb = h1[pl.ds(j * L, L)]
                s = ha + hb
                inc = plsc.cumsum(s)
                base = inc - s + carry
                o0[pl.ds(j * L, L)] = base
                o1[pl.ds(j * L, L)] = base + ha
                h0[pl.ds(j * L, L)] = zeros
                h1[pl.ds(j * L, L)] = zeros
                return carry + jnp.sum(s)

            def chain(i16, m, shift, nshift, dst, offs):
                dig = lax.shift_right_logical(m, shift) & MASK11 if shift else m & MASK11
                cnt, lastm = plsc.scan_count(dig)
                base = plsc.load_gather(offs, [dig])
                pos = base + cnt
                plsc.store_scatter(dst, [pos], i16)
                plsc.addupdate_scatter(offs, [dig], cnt, mask=lastm)
                if nshift is not None:
                    dig2 = lax.shift_right_logical(m, nshift)
                    if nshift < 22:
                        dig2 = dig2 & MASK11
                    lo = pos < HALF
                    hist_update(h0, dig2, mask=lo)
                    hist_update(h1, dig2, mask=jnp.logical_not(lo))

            lax.fori_loop(0, NB // L, scan_body, jnp.int32(-1))

            def p0_body(v, _):
                chain(v * L + iota, kbuf[pl.ds(v * L, L)], 0, 11, abuf, o0)
                vb = v + HVREGS
                chain(vb * L + iota, kbuf[pl.ds(vb * L, L)], 0, 11, abuf, o1)
                return 0

            lax.fori_loop(0, HVREGS, p0_body, 0)

            lax.fori_loop(0, NB // L, scan_body, jnp.int32(-1))

            def p1_body(v, _):
                iA = abuf[pl.ds(v * L, L)]
                chain(iA, plsc.load_gather(kbuf, [iA]), 11, 22, bbuf, o0)
                iB = abuf[pl.ds((v + HVREGS) * L, L)]
                chain(iB, plsc.load_gather(kbuf, [iB]), 11, 22, bbuf, o1)
                return 0

            lax.fori_loop(0, HVREGS, p1_body, 0)

            lax.fori_loop(0, NB // L, scan_body, jnp.int32(-1))

            def p2_body(v, _):
                iA = bbuf[pl.ds(v * L, L)]
                chain(iA, plsc.load_gather(kbuf, [iA]), 22, None, abuf, o0)
                iB = bbuf[pl.ds((v + HVREGS) * L, L)]
                chain(iB, plsc.load_gather(kbuf, [iB]), 22, None, abuf, o1)
                return 0

            lax.fori_loop(0, HVREGS, p2_body, 0)

            def fin_body(v, _):
                for vv in (v, v + HVREGS):
                    i16 = abuf[pl.ds(vv * L, L)]
                    m = plsc.load_gather(kbuf, [i16])
                    bbuf[pl.ds(vv * L, L)] = jnp.where(m < 0, m ^ MIN32, ~m)
                return 0

            lax.fori_loop(0, HVREGS, fin_body, 0)
            pltpu.sync_copy(bbuf, vals_hbm.at[row])
            pltpu.sync_copy(abuf, idx_hbm.at[row])
            return 0

        lax.fori_loop(0, rows_per_w, row_body, 0)

    return sort_kernel


@jax.jit
def kernel(input_tensors):
    xbits = lax.bitcast_convert_type(input_tensors, jnp.int32)
    vbits, idx = _build_sort_kernel()(xbits)
    values = lax.bitcast_convert_type(vbits, jnp.float32)
    return (values, idx)


# speedup vs baseline: 1.1767x; 1.1767x over previous
import functools
import jax
import jax.numpy as jnp
from jax import lax
from jax.experimental import pallas as pl
from jax.experimental.pallas import tpu as pltpu
from jax.experimental.pallas import tpu_sc as plsc

R = 1024
N = 32768
L = 16
NB = 2048
VREGS = N // L
MIN32 = jnp.int32(-2147483648)
MASK11 = jnp.int32(0x7FF)


@functools.cache
def _build_sort_kernel():
    info = plsc.get_sparse_core_info()
    nw = info.num_cores * info.num_subcores
    assert R % nw == 0
    rows_per_w = R // nw
    mesh = plsc.VectorSubcoreMesh(core_axis_name="c", subcore_axis_name="s")

    @functools.partial(
        pl.kernel,
        out_type=[
            jax.ShapeDtypeStruct((R, N), jnp.int32),
            jax.ShapeDtypeStruct((R, N), jnp.int32),
        ],
        mesh=mesh,
        compiler_params=pltpu.CompilerParams(needs_layout_passes=False),
        scratch_types=[
            pltpu.VMEM((N,), jnp.int32),
            pltpu.VMEM((N,), jnp.int32),
            pltpu.VMEM((N,), jnp.int32),
            pltpu.VMEM((NB,), jnp.int32),
            pltpu.VMEM((NB,), jnp.int32),
            pltpu.SemaphoreType.DMA,
            pltpu.SemaphoreType.DMA,
            pltpu.SemaphoreType.DMA,
        ],
    )
    def sort_kernel(xbits_hbm, vals_hbm, idx_hbm, kbuf, abuf, bbuf, hist, offs,
                    sem_in, sem_ov, sem_oi):
        wid = lax.axis_index("s") * info.num_cores + lax.axis_index("c")
        row0 = wid * rows_per_w
        iota = lax.iota(jnp.int32, L)
        zeros = jnp.zeros((L,), jnp.int32)

        def clear_body(j, _):
            hist[pl.ds(j * L, L)] = zeros
            return 0

        lax.fori_loop(0, NB // L, clear_body, 0)
        pltpu.async_copy(xbits_hbm.at[row0], kbuf, sem_in)

        def row_body(r, _):
            row = row0 + r
            pltpu.make_async_copy(xbits_hbm.at[row], kbuf, sem_in).wait()

            def conv_body(v, _):
                b = kbuf[pl.ds(v * L, L)]
                m = jnp.where(b >= 0, b ^ MIN32, ~b)
                kbuf[pl.ds(v * L, L)] = m
                dig = m & MASK11
                cnt, lastm = plsc.scan_count(dig)
                plsc.addupdate_scatter(hist, [dig], cnt, mask=lastm)
                return 0

            lax.fori_loop(0, VREGS, conv_body, 0)

            def scan_body(j, carry):
                h = hist[pl.ds(j * L, L)]
                inc = plsc.cumsum(h)
                offs[pl.ds(j * L, L)] = inc - h + carry
                hist[pl.ds(j * L, L)] = zeros
                return carry + jnp.sum(h)

            def permute(i16, m, shift, nshift, dst):
                dig = lax.shift_right_logical(m, shift) & MASK11 if shift else m & MASK11
                cnt, lastm = plsc.scan_count(dig)
                base = plsc.load_gather(offs, [dig])
                plsc.store_scatter(dst, [base + cnt], i16)
                plsc.addupdate_scatter(offs, [dig], cnt, mask=lastm)
                if nshift is not None:
                    dig2 = lax.shift_right_logical(m, nshift)
                    if nshift < 22:
                        dig2 = dig2 & MASK11
                    cnt2, lastm2 = plsc.scan_count(dig2)
                    plsc.addupdate_scatter(hist, [dig2], cnt2, mask=lastm2)

            lax.fori_loop(0, NB // L, scan_body, jnp.int32(-1))

            @pl.when(r > 0)
            def _():
                pltpu.make_async_copy(abuf, idx_hbm.at[row - 1], sem_oi).wait()

            def p0_body(v, _):
                permute(v * L + iota, kbuf[pl.ds(v * L, L)], 0, 11, abuf)
                return 0

            lax.fori_loop(0, VREGS, p0_body, 0)

            lax.fori_loop(0, NB // L, scan_body, jnp.int32(-1))

            @pl.when(r > 0)
            def _():
                pltpu.make_async_copy(bbuf, vals_hbm.at[row - 1], sem_ov).wait()

            def p1_body(v, _):
                i16 = abuf[pl.ds(v * L, L)]
                permute(i16, plsc.load_gather(kbuf, [i16]), 11, 22, bbuf)
                return 0

            lax.fori_loop(0, VREGS, p1_body, 0)

            lax.fori_loop(0, NB // L, scan_body, jnp.int32(-1))

            def p2_body(v, _):
                i16 = bbuf[pl.ds(v * L, L)]
                permute(i16, plsc.load_gather(kbuf, [i16]), 22, None, abuf)
                return 0

            lax.fori_loop(0, VREGS, p2_body, 0)

            def fin_body(v, _):
                i16 = abuf[pl.ds(v * L, L)]
                m = plsc.load_gather(kbuf, [i16])
                bbuf[pl.ds(v * L, L)] = jnp.where(m < 0, m ^ MIN32, ~m)
                return 0

            lax.fori_loop(0, VREGS, fin_body, 0)

            pltpu.async_copy(abuf, idx_hbm.at[row], sem_oi)
            pltpu.async_copy(bbuf, vals_hbm.at[row], sem_ov)

            @pl.when(r < rows_per_w - 1)
            def _():
                pltpu.async_copy(xbits_hbm.at[row + 1], kbuf, sem_in)

            return 0

        lax.fori_loop(0, rows_per_w, row_body, 0)
        last = row0 + rows_per_w - 1
        pltpu.make_async_copy(abuf, idx_hbm.at[last], sem_oi).wait()
        pltpu.make_async_copy(bbuf, vals_hbm.at[last], sem_ov).wait()

    return sort_kernel


@jax.jit
def kernel(input_tensors):
    xbits = lax.bitcast_convert_type(input_tensors, jnp.int32)
    vbits, idx = _build_sort_kernel()(xbits)
    values = lax.bitcast_convert_type(vbits, jnp.float32)
    return (values, idx)
